# Initial kernel scaffold; baseline (speedup 1.0000x reference)
import jax
import jax.numpy as jnp
from jax.experimental import pallas as pl


def kernel(x, fe_W1, fe_b1, fe_W2, fe_b2, proj_W, proj_b, codebooks, dec_W1, dec_b1, dec_W2, dec_b2):
    raise NotImplementedError("write your pallas kernel here")


# speedup vs baseline: 2.6375x; 2.6375x over previous
import functools

import jax
import jax.numpy as jnp
from jax.experimental import pallas as pl
from jax.experimental.pallas import tpu as pltpu


def _fused_body(x_ref, feW1_ref, feb1_ref, feW2_ref, feb2_ref, projW_ref,
                projb_ref, cb_ref, decW1_ref, decb1_ref, decW2_ref,
                decb2_ref, out_ref, *, n_cat, levels, feats, k_codes):
    f32 = jnp.float32
    x = x_ref[...]
    blk = x.shape[0]
    q_parts = []
    for c in range(n_cat):
        xc = x[:, c * feats:(c + 1) * feats]
        h = jnp.dot(xc, feW1_ref[c], preferred_element_type=f32)
        h = jnp.maximum(h + feb1_ref[c:c + 1, :], 0.0)
        emb = jnp.dot(h, feW2_ref[c], preferred_element_type=f32)
        emb = emb + feb2_ref[c:c + 1, :]
        for l in range(levels):
            z = jnp.dot(emb, projW_ref[c, l], preferred_element_type=f32)
            z = z + projb_ref[c, l:l + 1, :]
            cb = cb_ref[c, l]
            cross = jax.lax.dot_general(
                z, cb, (((1,), (1,)), ((), ())),
                preferred_element_type=f32)
            z2 = jnp.sum(z * z, axis=-1, keepdims=True)
            e2 = jnp.sum(cb * cb, axis=-1)
            dist = z2 - 2.0 * cross + e2[None, :]
            idx = jnp.argmin(dist, axis=-1)
            onehot = (jax.lax.broadcasted_iota(jnp.int32, (blk, k_codes), 1)
                      == idx[:, None]).astype(f32)
            q_parts.append(jnp.dot(onehot, cb, preferred_element_type=f32))
    flat = jnp.concatenate(q_parts, axis=1)
    h2 = jnp.dot(flat, decW1_ref[...], preferred_element_type=f32)
    h2 = jnp.maximum(h2 + decb1_ref[...], 0.0)
    out = jnp.dot(h2, decW2_ref[...], preferred_element_type=f32)
    out_ref[...] = out + decb2_ref[...]


def kernel(x, fe_W1, fe_b1, fe_W2, fe_b2, proj_W, proj_b, codebooks,
           dec_W1, dec_b1, dec_W2, dec_b2):
    bsz, in_dim = x.shape
    n_cat, feats, hid = fe_W1.shape
    emb_d = fe_W2.shape[2]
    levels, k_codes, d = codebooks.shape[1], codebooks.shape[2], codebooks.shape[3]
    blk = 512
    grid = (bsz // blk,)

    def rep(shape):
        return pl.BlockSpec(shape, lambda i: (0,) * len(shape))

    body = functools.partial(_fused_body, n_cat=n_cat, levels=levels,
                             feats=feats, k_codes=k_codes)
    return pl.pallas_call(
        body,
        grid=grid,
        in_specs=[
            pl.BlockSpec((blk, in_dim), lambda i: (i, 0)),
            rep(fe_W1.shape), rep(fe_b1.shape),
            rep(fe_W2.shape), rep(fe_b2.shape),
            rep(proj_W.shape), rep(proj_b.shape),
            rep(codebooks.shape),
            rep(dec_W1.shape), rep((1, dec_b1.shape[0])),
            rep(dec_W2.shape), rep((1, dec_b2.shape[0])),
        ],
        out_specs=pl.BlockSpec((blk, in_dim), lambda i: (i, 0)),
        out_shape=jax.ShapeDtypeStruct((bsz, in_dim), jnp.float32),
    )(x, fe_W1, fe_b1, fe_W2, fe_b2, proj_W, proj_b, codebooks,
      dec_W1, dec_b1.reshape(1, -1), dec_W2, dec_b2.reshape(1, -1))
